# baseline (speedup 1.0000x reference)
import functools

import jax
import jax.numpy as jnp
from jax import lax
from jax.experimental import pallas as pl
from jax.experimental.pallas import tpu as pltpu
from jax.experimental.pallas import tpu_sc as plsc

_BATCH = 1024
_EMB = 16
_HID = 64
_NBR = 6
_NBLK = 2048

_NC = 2
_NS = 16
_NW = _NC * _NS
_CH = 96


def _sc_gather(br_table, map_table, br_idx, m_idx):
    b1 = br_idx.shape[0]
    b2 = m_idx.shape[0]
    c1 = b1 // _NW
    c2 = b2 // _NW
    nch = c1 // _CH

    mesh = plsc.VectorSubcoreMesh(core_axis_name="c", subcore_axis_name="s")

    @functools.partial(
        pl.kernel,
        mesh=mesh,
        out_type=(
            jax.ShapeDtypeStruct((b1, _EMB), jnp.float32),
            jax.ShapeDtypeStruct((b2, _EMB), jnp.float32),
        ),
        scratch_types=[
            pltpu.VMEM((_CH,), jnp.int32),
            pltpu.VMEM((_CH, _EMB), jnp.float32),
            pltpu.VMEM((c2,), jnp.int32),
            pltpu.VMEM((c2, _EMB), jnp.float32),
            pltpu.SemaphoreType.DMA,
        ],
        compiler_params=pltpu.CompilerParams(use_tc_tiling_on_sc=False),
    )
    def gather_kernel(brt_hbm, mpt_hbm, bidx_hbm, midx_hbm, out_br, out_mp,
                      idx_v, rows_v, midx_v, mrows_v, sem):
        wid = lax.axis_index("s") * _NC + lax.axis_index("c")
        base = wid * c1
        for c in range(nch):
            off = base + c * _CH
            pltpu.sync_copy(bidx_hbm.at[pl.ds(off, _CH)], idx_v)
            pltpu.async_copy(brt_hbm.at[idx_v], rows_v, sem).wait()
            pltpu.sync_copy(rows_v, out_br.at[pl.ds(off, _CH)])
        base2 = wid * c2
        pltpu.sync_copy(midx_hbm.at[pl.ds(base2, c2)], midx_v)
        pltpu.async_copy(mpt_hbm.at[midx_v], mrows_v, sem).wait()
        pltpu.sync_copy(mrows_v, out_mp.at[pl.ds(base2, c2)])

    return gather_kernel(br_table, map_table, br_idx, m_idx)


def _mlp_body(xbr_ref, xmap_ref, w1a_ref, w1b_ref, b1_ref, w2_ref, b2_ref,
              out_ref, h_ref):
    @pl.when(pl.program_id(0) == 0)
    def _():
        acc = jnp.dot(xbr_ref[...], w1a_ref[...],
                      preferred_element_type=jnp.float32)
        acc = acc + jnp.dot(xmap_ref[...], w1b_ref[...],
                            preferred_element_type=jnp.float32)
        h_ref[...] = jnp.maximum(acc + b1_ref[...], 0.0)

    out_ref[...] = (
        jnp.dot(h_ref[...], w2_ref[...], preferred_element_type=jnp.float32)
        + b2_ref[...]
    )


def kernel(friends, enemies, map_idx, brawler_table, map_table, W1, b1, W2, b2):
    n_out = W2.shape[1]
    br_idx = jnp.concatenate([friends, enemies], axis=1).reshape(-1)
    m_idx = map_idx.reshape(-1)

    br_rows, xmap = _sc_gather(brawler_table, map_table, br_idx, m_idx)
    xbr = br_rows.reshape(_BATCH, _NBR * _EMB)

    w1a = W1[: _NBR * _EMB]
    w1b = W1[_NBR * _EMB:]
    b1r = b1.reshape(1, _HID)
    b2r = b2.reshape(1, n_out)

    grid = pl.cdiv(n_out, _NBLK)
    return pl.pallas_call(
        _mlp_body,
        grid=(grid,),
        in_specs=[
            pl.BlockSpec((_BATCH, _NBR * _EMB), lambda j: (0, 0)),
            pl.BlockSpec((_BATCH, _EMB), lambda j: (0, 0)),
            pl.BlockSpec((_NBR * _EMB, _HID), lambda j: (0, 0)),
            pl.BlockSpec((_EMB, _HID), lambda j: (0, 0)),
            pl.BlockSpec((1, _HID), lambda j: (0, 0)),
            pl.BlockSpec((_HID, _NBLK), lambda j: (0, j)),
            pl.BlockSpec((1, _NBLK), lambda j: (0, j)),
        ],
        out_specs=pl.BlockSpec((_BATCH, _NBLK), lambda j: (0, j)),
        out_shape=jax.ShapeDtypeStruct((_BATCH, n_out), jnp.float32),
        scratch_shapes=[pltpu.VMEM((_BATCH, _HID), jnp.float32)],
        compiler_params=pltpu.CompilerParams(
            dimension_semantics=("arbitrary",)),
    )(xbr, xmap, w1a, w1b, b1r, W2, b2r)


# speedup vs baseline: 2.7926x; 2.7926x over previous
import functools

import jax
import jax.numpy as jnp
from jax import lax
from jax.experimental import pallas as pl
from jax.experimental.pallas import tpu as pltpu
from jax.experimental.pallas import tpu_sc as plsc

_BATCH = 1024
_EMB = 16
_HID = 64
_NBR = 6
_XCOL = 128
_NBLK = 2048

_NC = 2
_NS = 16
_NW = _NC * _NS
_BPW = _BATCH // _NW


def _sc_gather_x(br_table, map_table, br_idx_sm, m_idx):
    mesh = plsc.VectorSubcoreMesh(core_axis_name="c", subcore_axis_name="s")

    @functools.partial(
        pl.kernel,
        mesh=mesh,
        out_type=jax.ShapeDtypeStruct((_BATCH, _XCOL), jnp.float32),
        scratch_types=[
            pltpu.VMEM((_NBR + 1, _BPW), jnp.int32),
            pltpu.VMEM(((_NBR + 1) * _BPW, _EMB), jnp.float32),
            pltpu.VMEM((_BPW, _XCOL), jnp.float32),
            pltpu.SemaphoreType.DMA,
        ],
        compiler_params=pltpu.CompilerParams(use_tc_tiling_on_sc=False),
    )
    def gather_kernel(brt_hbm, mpt_hbm, bidx_hbm, midx_hbm, out_x,
                      idxs_v, rows_v, x_v, sem):
        wid = lax.axis_index("s") * _NC + lax.axis_index("c")
        base = wid * _BPW
        for k in range(_NBR):
            pltpu.sync_copy(bidx_hbm.at[k, pl.ds(base, _BPW)], idxs_v.at[k])
        pltpu.sync_copy(midx_hbm.at[pl.ds(base, _BPW)], idxs_v.at[_NBR])
        cps = [
            pltpu.async_copy(brt_hbm.at[idxs_v.at[k]],
                             rows_v.at[pl.ds(k * _BPW, _BPW)], sem)
            for k in range(_NBR)
        ]
        cps.append(
            pltpu.async_copy(mpt_hbm.at[idxs_v.at[_NBR]],
                             rows_v.at[pl.ds(_NBR * _BPW, _BPW)], sem))
        for cp in cps:
            cp.wait()
        zero = jnp.zeros((_EMB,), jnp.float32)
        for r in range(_BPW):
            for k in range(_NBR + 1):
                x_v[r, pl.ds(k * _EMB, _EMB)] = rows_v[k * _BPW + r, :]
            x_v[r, pl.ds((_NBR + 1) * _EMB, _EMB)] = zero
        pltpu.sync_copy(x_v, out_x.at[pl.ds(base, _BPW)])

    return gather_kernel(br_table, map_table, br_idx_sm, m_idx)


def _mlp_body(x_ref, w1p_ref, b1_ref, w2_ref, b2_ref, out_ref, ht_ref):
    @pl.when(pl.program_id(0) == 0)
    def _():
        ht = lax.dot_general(w1p_ref[...], x_ref[...],
                             (((0,), (1,)), ((), ())),
                             preferred_element_type=jnp.float32)
        ht_ref[...] = jnp.maximum(ht + b1_ref[...], 0.0)

    out_ref[...] = (
        lax.dot_general(w2_ref[...], ht_ref[...],
                        (((0,), (0,)), ((), ())),
                        preferred_element_type=jnp.float32)
        + lax.dot_general(b2_ref[...], jnp.ones((1, _BATCH), jnp.float32),
                          (((0,), (0,)), ((), ())),
                          preferred_element_type=jnp.float32)
    )


def kernel(friends, enemies, map_idx, brawler_table, map_table, W1, b1, W2, b2):
    n_out = W2.shape[1]
    br_idx_sm = jnp.concatenate([friends, enemies], axis=1).T
    m_idx = map_idx.reshape(-1)

    x = _sc_gather_x(brawler_table, map_table, br_idx_sm, m_idx)

    w1p = jnp.pad(W1, ((0, _XCOL - W1.shape[0]), (0, 0)))
    b1c = b1.reshape(_HID, 1)
    b2r = b2.reshape(1, n_out)

    grid = pl.cdiv(n_out, _NBLK)
    out_t = pl.pallas_call(
        _mlp_body,
        grid=(grid,),
        in_specs=[
            pl.BlockSpec((_BATCH, _XCOL), lambda j: (0, 0)),
            pl.BlockSpec((_XCOL, _HID), lambda j: (0, 0)),
            pl.BlockSpec((_HID, 1), lambda j: (0, 0)),
            pl.BlockSpec((_HID, _NBLK), lambda j: (0, j)),
            pl.BlockSpec((1, _NBLK), lambda j: (0, j)),
        ],
        out_specs=pl.BlockSpec((_NBLK, _BATCH), lambda j: (j, 0)),
        out_shape=jax.ShapeDtypeStruct((n_out, _BATCH), jnp.float32),
        scratch_shapes=[pltpu.VMEM((_HID, _BATCH), jnp.float32)],
        compiler_params=pltpu.CompilerParams(
            dimension_semantics=("arbitrary",)),
    )(x, w1p, b1c, W2, b2r)
    return out_t.T
